# Initial kernel scaffold; baseline (speedup 1.0000x reference)
import jax
import jax.numpy as jnp
from jax.experimental import pallas as pl


def kernel(x, w_gate, fc1, b1, fc2, b2):
    raise NotImplementedError("write your pallas kernel here")


# speedup vs baseline: 3.4916x; 3.4916x over previous
import functools

import jax
import jax.numpy as jnp
import numpy as np
from jax import lax
from jax.experimental import pallas as pl
from jax.experimental.pallas import tpu as pltpu
from jax.experimental.pallas import tpu_sc as plsc

N_TOK = 4096
D_MODEL = 1024
HID = 2048
NE = 8
LANES = 128
BM = 256
NBLK = N_TOK * 2 // BM + NE
SPAD = NBLK * BM
CHUNK = 512
NCHUNK = N_TOK // CHUNK
SCCH = 32
_EPS = float(np.finfo(float).eps)


def _routing_body(x_ref, wg_ref, pos_ref, g_ref, binfo_ref, loss_ref, i1_s, i2_s):
    logits = jnp.dot(x_ref[...], wg_ref[...], preferred_element_type=jnp.float32)
    lane = lax.broadcasted_iota(jnp.int32, (N_TOK, LANES), 1)
    neg = jnp.float32(-1e30)
    lv = jnp.where(lane < NE, logits, neg)
    m1 = jnp.max(lv, axis=1, keepdims=True)
    i1 = jnp.min(jnp.where(lv == m1, lane, LANES), axis=1, keepdims=True)
    lv2 = jnp.where(lane == i1, neg, lv)
    m2 = jnp.max(lv2, axis=1, keepdims=True)
    i2 = jnp.min(jnp.where(lv2 == m2, lane, LANES), axis=1, keepdims=True)
    e2 = jnp.exp(m2 - m1)
    g1 = 1.0 / (1.0 + e2)
    g2 = e2 / (1.0 + e2)
    g_ref[:, 0:1] = g1
    g_ref[:, 1:2] = g2
    i1_s[...] = i1
    i2_s[...] = i2

    oh1 = jnp.where(lane == i1, 1.0, 0.0)
    oh2 = jnp.where(lane == i2, 1.0, 0.0)
    counts = jnp.sum(oh1 + oh2, axis=0, keepdims=True)
    imp = jnp.sum(oh1 * g1 + oh2 * g2, axis=0, keepdims=True)
    lane_r = lax.broadcasted_iota(jnp.int32, (1, LANES), 1)
    v8 = jnp.where(lane_r < NE, 1.0, 0.0)

    def cv2(v):
        mean = jnp.sum(v * v8, axis=1, keepdims=True) / NE
        var = jnp.sum(((v - mean) ** 2) * v8, axis=1, keepdims=True) / (NE - 1)
        return var / (mean * mean + 1e-10)

    loss_ref[...] = (cv2(imp) + cv2(counts)) * 1e-2

    nb_f = jnp.floor((counts + (BM - 1)) * (1.0 / BM))
    row_i = lax.broadcasted_iota(jnp.int32, (LANES, LANES), 0)
    col_i = lax.broadcasted_iota(jnp.int32, (LANES, LANES), 1)
    tri = jnp.where(row_i < col_i, 1.0, 0.0)
    blk_start = jnp.dot(nb_f, tri, preferred_element_type=jnp.float32)
    seg_start = blk_start * BM
    blk_end = blk_start + nb_f
    ident = jnp.where(row_i == col_i, 1.0, 0.0)
    blk_end_col = jnp.sum(ident * blk_end, axis=1, keepdims=True)
    amat = jnp.where(
        (blk_end_col <= col_i.astype(jnp.float32)) & (row_i < NE), 1.0, 0.0)
    be = jnp.sum(amat, axis=0, keepdims=True)
    last_e = jnp.max(jnp.where((nb_f > 0) & (lane_r < NE),
                               lane_r.astype(jnp.float32), 0.0))
    be = jnp.minimum(be, last_e)
    nblocks = jnp.sum(nb_f * v8)
    binfo = jnp.where(lane_r == NBLK, nblocks, be)
    binfo_ref[...] = binfo.astype(jnp.int32)

    lsr = lax.broadcasted_iota(jnp.int32, (CHUNK, CHUNK), 0)
    lsc = lax.broadcasted_iota(jnp.int32, (CHUNK, CHUNK), 1)
    ltri = jnp.where(lsc < lsr, 1.0, 0.0)
    lane_c = lax.broadcasted_iota(jnp.int32, (CHUNK, LANES), 1)

    def body(c, base):
        rs = c * CHUNK
        i1c = i1_s[pl.ds(rs, CHUNK), :]
        i2c = i2_s[pl.ds(rs, CHUNK), :]
        oh1c = jnp.where(lane_c == i1c, 1.0, 0.0)
        oh2c = jnp.where(lane_c == i2c, 1.0, 0.0)
        oc = oh1c + oh2c
        cs = jnp.dot(ltri, oc, preferred_element_type=jnp.float32) + base
        p0 = jnp.sum((cs + seg_start) * oh1c, axis=1, keepdims=True)
        p1 = jnp.sum((cs + seg_start) * oh2c, axis=1, keepdims=True)
        pos_ref[pl.ds(rs, CHUNK), 0:1] = p0.astype(jnp.int32)
        pos_ref[pl.ds(rs, CHUNK), 1:2] = p1.astype(jnp.int32)
        return base + jnp.sum(oc, axis=0, keepdims=True)

    lax.fori_loop(0, NCHUNK, body, jnp.zeros((1, LANES), jnp.float32))


def _routing(x, wg_pad):
    return pl.pallas_call(
        _routing_body,
        out_shape=(
            jax.ShapeDtypeStruct((N_TOK, 2), jnp.int32),
            jax.ShapeDtypeStruct((N_TOK, 2), jnp.float32),
            jax.ShapeDtypeStruct((1, LANES), jnp.int32),
            jax.ShapeDtypeStruct((1, 1), jnp.float32),
        ),
        scratch_shapes=[
            pltpu.VMEM((N_TOK, 1), jnp.int32),
            pltpu.VMEM((N_TOK, 1), jnp.int32),
        ],
    )(x, wg_pad)


def _dispatch(x, p0, p1):
    info = plsc.get_sparse_core_info()
    nw = info.num_cores * info.num_subcores
    tpw = N_TOK // nw
    mesh = plsc.VectorSubcoreMesh(core_axis_name="c", subcore_axis_name="s")

    @functools.partial(
        pl.kernel,
        out_type=jax.ShapeDtypeStruct((SPAD, D_MODEL), jnp.float32),
        mesh=mesh,
        scratch_types=[
            pltpu.VMEM((SCCH, D_MODEL), jnp.float32),
            pltpu.VMEM((SCCH,), jnp.int32),
            pltpu.VMEM((SCCH,), jnp.int32),
            pltpu.SemaphoreType.DMA,
        ],
    )
    def k(x_hbm, p0_hbm, p1_hbm, out_hbm, xbuf, idx0, idx1, sem):
        wid = lax.axis_index("s") * info.num_cores + lax.axis_index("c")
        for j in range(tpw // SCCH):
            base = wid * tpw + j * SCCH
            pltpu.sync_copy(x_hbm.at[pl.ds(base, SCCH)], xbuf)
            pltpu.sync_copy(p0_hbm.at[pl.ds(base, SCCH)], idx0)
            pltpu.sync_copy(p1_hbm.at[pl.ds(base, SCCH)], idx1)
            pltpu.async_copy(xbuf, out_hbm.at[idx0], sem).wait()
            pltpu.async_copy(xbuf, out_hbm.at[idx1], sem).wait()

    return k(x, p0, p1)


def _gelu_exact(h):
    return 0.5 * h * (1.0 + lax.erf(h * np.float32(1.0 / np.sqrt(2.0))))


def _gmm_body(s_ref, x_ref, fc1_ref, b1_ref, fc2_ref, b2_ref, o_ref):
    b = pl.program_id(0)

    @pl.when(b < s_ref[NBLK])
    def _():
        xb = x_ref[...].astype(jnp.bfloat16)
        h = jnp.dot(xb, fc1_ref[0], preferred_element_type=jnp.float32)
        h = _gelu_exact(h + b1_ref[0])
        o = jnp.dot(h.astype(jnp.bfloat16), fc2_ref[0],
                    preferred_element_type=jnp.float32)
        o_ref[...] = o + b2_ref[0]


def _gmm(binfo, x_sorted, fc1, b1, fc2, b2):
    grid_spec = pltpu.PrefetchScalarGridSpec(
        num_scalar_prefetch=1,
        grid=(NBLK,),
        in_specs=[
            pl.BlockSpec((BM, D_MODEL), lambda b, s: (b, 0)),
            pl.BlockSpec((1, D_MODEL, HID), lambda b, s: (s[b], 0, 0)),
            pl.BlockSpec((1, 1, HID), lambda b, s: (s[b], 0, 0)),
            pl.BlockSpec((1, HID, D_MODEL), lambda b, s: (s[b], 0, 0)),
            pl.BlockSpec((1, 1, D_MODEL), lambda b, s: (s[b], 0, 0)),
        ],
        out_specs=pl.BlockSpec((BM, D_MODEL), lambda b, s: (b, 0)),
    )
    return pl.pallas_call(
        _gmm_body,
        grid_spec=grid_spec,
        out_shape=jax.ShapeDtypeStruct((SPAD, D_MODEL), jnp.float32),
    )(binfo, x_sorted, fc1, b1, fc2, b2)


def _combine_gather(v, p0, p1):
    info = plsc.get_sparse_core_info()
    nw = info.num_cores * info.num_subcores
    tpw = N_TOK // nw
    mesh = plsc.VectorSubcoreMesh(core_axis_name="c", subcore_axis_name="s")

    @functools.partial(
        pl.kernel,
        out_type=(jax.ShapeDtypeStruct((N_TOK, D_MODEL), jnp.float32),
                  jax.ShapeDtypeStruct((N_TOK, D_MODEL), jnp.float32)),
        mesh=mesh,
        scratch_types=[
            pltpu.VMEM((SCCH, D_MODEL), jnp.float32),
            pltpu.VMEM((SCCH,), jnp.int32),
            pltpu.SemaphoreType.DMA,
        ],
    )
    def k(v_hbm, p0_hbm, p1_hbm, o0_hbm, o1_hbm, buf, idx, sem):
        wid = lax.axis_index("s") * info.num_cores + lax.axis_index("c")
        for j in range(tpw // SCCH):
            base = wid * tpw + j * SCCH
            pltpu.sync_copy(p0_hbm.at[pl.ds(base, SCCH)], idx)
            pltpu.async_copy(v_hbm.at[idx], buf, sem).wait()
            pltpu.sync_copy(buf, o0_hbm.at[pl.ds(base, SCCH)])
            pltpu.sync_copy(p1_hbm.at[pl.ds(base, SCCH)], idx)
            pltpu.async_copy(v_hbm.at[idx], buf, sem).wait()
            pltpu.sync_copy(buf, o1_hbm.at[pl.ds(base, SCCH)])

    return k(v, p0, p1)


def _combine_body(v0_ref, v1_ref, g_ref, y_ref):
    g0 = g_ref[:, 0:1]
    g1 = g_ref[:, 1:2]
    s = g0 * jnp.exp(v0_ref[...]) + g1 * jnp.exp(v1_ref[...])
    s = jnp.where(s == 0.0, jnp.float32(_EPS), s)
    y_ref[...] = jnp.log(s)


def _combine(v0, v1, g):
    rb = 512
    return pl.pallas_call(
        _combine_body,
        grid=(N_TOK // rb,),
        in_specs=[
            pl.BlockSpec((rb, D_MODEL), lambda i: (i, 0)),
            pl.BlockSpec((rb, D_MODEL), lambda i: (i, 0)),
            pl.BlockSpec((rb, 2), lambda i: (i, 0)),
        ],
        out_specs=pl.BlockSpec((rb, D_MODEL), lambda i: (i, 0)),
        out_shape=jax.ShapeDtypeStruct((N_TOK, D_MODEL), jnp.float32),
    )(v0, v1, g)


def kernel(x, w_gate, fc1, b1, fc2, b2):
    wg_pad = jnp.pad(w_gate, ((0, 0), (0, LANES - NE)))
    pos, g, binfo, loss = _routing(x, wg_pad)
    p0 = pos[:, 0]
    p1 = pos[:, 1]
    x_sorted = _dispatch(x, p0, p1)
    v = _gmm(binfo[0], x_sorted,
             fc1.astype(jnp.bfloat16), b1[:, None, :],
             fc2.astype(jnp.bfloat16), b2[:, None, :])
    v0, v1 = _combine_gather(v, p0, p1)
    y = _combine(v0, v1, g)
    return y, loss[0, 0]


# speedup vs baseline: 3.5857x; 1.0269x over previous
import functools

import jax
import jax.numpy as jnp
import numpy as np
from jax import lax
from jax.experimental import pallas as pl
from jax.experimental.pallas import tpu as pltpu
from jax.experimental.pallas import tpu_sc as plsc

N_TOK = 4096
D_MODEL = 1024
HID = 2048
NE = 8
LANES = 128
BM = 256
NBLK = N_TOK * 2 // BM + NE
SPAD = NBLK * BM
CHUNK = 512
NCHUNK = N_TOK // CHUNK
SCCH = 32
_EPS = float(np.finfo(float).eps)


def _routing_body(x_ref, wg_ref, pos_ref, g_ref, binfo_ref, loss_ref, i1_s, i2_s):
    logits = jnp.dot(x_ref[...], wg_ref[...], preferred_element_type=jnp.float32)
    lane = lax.broadcasted_iota(jnp.int32, (N_TOK, LANES), 1)
    neg = jnp.float32(-1e30)
    lv = jnp.where(lane < NE, logits, neg)
    m1 = jnp.max(lv, axis=1, keepdims=True)
    i1 = jnp.min(jnp.where(lv == m1, lane, LANES), axis=1, keepdims=True)
    lv2 = jnp.where(lane == i1, neg, lv)
    m2 = jnp.max(lv2, axis=1, keepdims=True)
    i2 = jnp.min(jnp.where(lv2 == m2, lane, LANES), axis=1, keepdims=True)
    e2 = jnp.exp(m2 - m1)
    g1 = 1.0 / (1.0 + e2)
    g2 = e2 / (1.0 + e2)
    g_ref[:, 0:1] = g1
    g_ref[:, 1:2] = g2
    i1_s[...] = i1
    i2_s[...] = i2

    oh1 = jnp.where(lane == i1, 1.0, 0.0)
    oh2 = jnp.where(lane == i2, 1.0, 0.0)
    counts = jnp.sum(oh1 + oh2, axis=0, keepdims=True)
    imp = jnp.sum(oh1 * g1 + oh2 * g2, axis=0, keepdims=True)
    lane_r = lax.broadcasted_iota(jnp.int32, (1, LANES), 1)
    v8 = jnp.where(lane_r < NE, 1.0, 0.0)

    def cv2(v):
        mean = jnp.sum(v * v8, axis=1, keepdims=True) / NE
        var = jnp.sum(((v - mean) ** 2) * v8, axis=1, keepdims=True) / (NE - 1)
        return var / (mean * mean + 1e-10)

    loss_ref[...] = (cv2(imp) + cv2(counts)) * 1e-2

    nb_f = jnp.floor((counts + (BM - 1)) * (1.0 / BM))
    row_i = lax.broadcasted_iota(jnp.int32, (LANES, LANES), 0)
    col_i = lax.broadcasted_iota(jnp.int32, (LANES, LANES), 1)
    tri = jnp.where(row_i < col_i, 1.0, 0.0)
    blk_start = jnp.dot(nb_f, tri, preferred_element_type=jnp.float32)
    seg_start = blk_start * BM
    blk_end = blk_start + nb_f
    ident = jnp.where(row_i == col_i, 1.0, 0.0)
    blk_end_col = jnp.sum(ident * blk_end, axis=1, keepdims=True)
    amat = jnp.where(
        (blk_end_col <= col_i.astype(jnp.float32)) & (row_i < NE), 1.0, 0.0)
    be = jnp.sum(amat, axis=0, keepdims=True)
    last_e = jnp.max(jnp.where((nb_f > 0) & (lane_r < NE),
                               lane_r.astype(jnp.float32), 0.0))
    be = jnp.minimum(be, last_e)
    nblocks = jnp.sum(nb_f * v8)
    binfo = jnp.where(lane_r == NBLK, nblocks, be)
    binfo_ref[...] = binfo.astype(jnp.int32)

    lsr = lax.broadcasted_iota(jnp.int32, (CHUNK, CHUNK), 0)
    lsc = lax.broadcasted_iota(jnp.int32, (CHUNK, CHUNK), 1)
    ltri = jnp.where(lsc < lsr, 1.0, 0.0)
    lane_c = lax.broadcasted_iota(jnp.int32, (CHUNK, LANES), 1)

    def body(c, base):
        rs = c * CHUNK
        i1c = i1_s[pl.ds(rs, CHUNK), :]
        i2c = i2_s[pl.ds(rs, CHUNK), :]
        oh1c = jnp.where(lane_c == i1c, 1.0, 0.0)
        oh2c = jnp.where(lane_c == i2c, 1.0, 0.0)
        oc = oh1c + oh2c
        cs = jnp.dot(ltri, oc, preferred_element_type=jnp.float32) + base
        p0 = jnp.sum((cs + seg_start) * oh1c, axis=1, keepdims=True)
        p1 = jnp.sum((cs + seg_start) * oh2c, axis=1, keepdims=True)
        pos_ref[pl.ds(rs, CHUNK), 0:1] = p0.astype(jnp.int32)
        pos_ref[pl.ds(rs, CHUNK), 1:2] = p1.astype(jnp.int32)
        return base + jnp.sum(oc, axis=0, keepdims=True)

    lax.fori_loop(0, NCHUNK, body, jnp.zeros((1, LANES), jnp.float32))


def _routing(x, wg_pad):
    return pl.pallas_call(
        _routing_body,
        out_shape=(
            jax.ShapeDtypeStruct((N_TOK, 2), jnp.int32),
            jax.ShapeDtypeStruct((N_TOK, 2), jnp.float32),
            jax.ShapeDtypeStruct((1, LANES), jnp.int32),
            jax.ShapeDtypeStruct((1, 1), jnp.float32),
        ),
        scratch_shapes=[
            pltpu.VMEM((N_TOK, 1), jnp.int32),
            pltpu.VMEM((N_TOK, 1), jnp.int32),
        ],
    )(x, wg_pad)


def _dispatch(x, p0_2d, p1_2d):
    info = plsc.get_sparse_core_info()
    nw = info.num_cores * info.num_subcores
    tpw = N_TOK // nw
    nch = tpw // SCCH
    mesh = plsc.VectorSubcoreMesh(core_axis_name="c", subcore_axis_name="s")

    @functools.partial(
        pl.kernel,
        out_type=jax.ShapeDtypeStruct((SPAD, D_MODEL), jnp.float32),
        mesh=mesh,
        scratch_types=[
            pltpu.VMEM((2, SCCH, D_MODEL), jnp.float32),
            pltpu.VMEM((nch, SCCH), jnp.int32),
            pltpu.VMEM((nch, SCCH), jnp.int32),
            pltpu.SemaphoreType.DMA,
            pltpu.SemaphoreType.DMA,
        ],
    )
    def k(x_hbm, p0_hbm, p1_hbm, out_hbm, xb, i0, i1, ld_sem, sc_sem):
        wid = lax.axis_index("s") * info.num_cores + lax.axis_index("c")
        pltpu.sync_copy(p0_hbm.at[pl.ds(wid * nch, nch)], i0)
        pltpu.sync_copy(p1_hbm.at[pl.ds(wid * nch, nch)], i1)
        pltpu.async_copy(
            x_hbm.at[pl.ds(wid * tpw, SCCH)], xb.at[0], ld_sem).wait()
        nxt = None
        for j in range(nch):
            cur = j % 2
            if j + 1 < nch:
                nxt = pltpu.async_copy(
                    x_hbm.at[pl.ds(wid * tpw + (j + 1) * SCCH, SCCH)],
                    xb.at[1 - cur], ld_sem)
            s0 = pltpu.async_copy(xb.at[cur], out_hbm.at[i0.at[j]], sc_sem)
            s1 = pltpu.async_copy(xb.at[cur], out_hbm.at[i1.at[j]], sc_sem)
            s0.wait()
            s1.wait()
            if j + 1 < nch:
                nxt.wait()

    return k(x, p0_2d, p1_2d)


def _gelu_exact(h):
    return 0.5 * h * (1.0 + lax.erf(h * np.float32(1.0 / np.sqrt(2.0))))


def _gmm_body(s_ref, x_ref, fc1_ref, b1_ref, fc2_ref, b2_ref, o_ref):
    b = pl.program_id(0)

    @pl.when(b < s_ref[NBLK])
    def _():
        xb = x_ref[...].astype(jnp.bfloat16)
        h = jnp.dot(xb, fc1_ref[0], preferred_element_type=jnp.float32)
        h = _gelu_exact(h + b1_ref[0])
        o = jnp.dot(h.astype(jnp.bfloat16), fc2_ref[0],
                    preferred_element_type=jnp.float32)
        o_ref[...] = o + b2_ref[0]


def _gmm(binfo, x_sorted, fc1, b1, fc2, b2):
    grid_spec = pltpu.PrefetchScalarGridSpec(
        num_scalar_prefetch=1,
        grid=(NBLK,),
        in_specs=[
            pl.BlockSpec((BM, D_MODEL),
                         lambda b, s: (jnp.minimum(b, s[NBLK] - 1), 0)),
            pl.BlockSpec((1, D_MODEL, HID), lambda b, s: (s[b], 0, 0)),
            pl.BlockSpec((1, 1, HID), lambda b, s: (s[b], 0, 0)),
            pl.BlockSpec((1, HID, D_MODEL), lambda b, s: (s[b], 0, 0)),
            pl.BlockSpec((1, 1, D_MODEL), lambda b, s: (s[b], 0, 0)),
        ],
        out_specs=pl.BlockSpec((BM, D_MODEL),
                               lambda b, s: (jnp.minimum(b, s[NBLK] - 1), 0)),
    )
    return pl.pallas_call(
        _gmm_body,
        grid_spec=grid_spec,
        out_shape=jax.ShapeDtypeStruct((SPAD, D_MODEL), jnp.float32),
    )(binfo, x_sorted, fc1, b1, fc2, b2)


def _combine_gather(v, p0, p1):
    info = plsc.get_sparse_core_info()
    nw = info.num_cores * info.num_subcores
    tpw = N_TOK // nw
    mesh = plsc.VectorSubcoreMesh(core_axis_name="c", subcore_axis_name="s")

    nch = tpw // SCCH

    @functools.partial(
        pl.kernel,
        out_type=(jax.ShapeDtypeStruct((N_TOK, D_MODEL), jnp.float32),
                  jax.ShapeDtypeStruct((N_TOK, D_MODEL), jnp.float32)),
        mesh=mesh,
        scratch_types=[
            pltpu.VMEM((2, SCCH, D_MODEL), jnp.float32),
            pltpu.VMEM((nch, SCCH), jnp.int32),
            pltpu.VMEM((nch, SCCH), jnp.int32),
            pltpu.SemaphoreType.DMA,
            pltpu.SemaphoreType.DMA,
            pltpu.SemaphoreType.DMA,
            pltpu.SemaphoreType.DMA,
        ],
    )
    def k(v_hbm, p0_hbm, p1_hbm, o0_hbm, o1_hbm, gb, i0, i1,
          ga_sem, gb_sem, sa_sem, sb_sem):
        wid = lax.axis_index("s") * info.num_cores + lax.axis_index("c")
        pltpu.sync_copy(p0_hbm.at[pl.ds(wid * nch, nch)], i0)
        pltpu.sync_copy(p1_hbm.at[pl.ds(wid * nch, nch)], i1)
        g_sems = (ga_sem, gb_sem)
        s_sems = (sa_sem, sb_sem)
        units = ([(i0, o0_hbm, j) for j in range(nch)]
                 + [(i1, o1_hbm, j) for j in range(nch)])
        nu = len(units)
        gathers = [None, None]
        stores = [None, None]
        idx0, _, _ = units[0]
        gathers[0] = pltpu.async_copy(v_hbm.at[idx0.at[0]], gb.at[0], g_sems[0])
        for u in range(nu):
            cur = u % 2
            nxt = 1 - cur
            if u + 1 < nu:
                if stores[nxt] is not None:
                    stores[nxt].wait()
                idxn, _, jn = units[u + 1]
                gathers[nxt] = pltpu.async_copy(
                    v_hbm.at[idxn.at[jn]], gb.at[nxt], g_sems[nxt])
            gathers[cur].wait()
            _, outr, j = units[u]
            stores[cur] = pltpu.async_copy(
                gb.at[cur], outr.at[pl.ds(wid * tpw + j * SCCH, SCCH)],
                s_sems[cur])
        stores[0].wait()
        stores[1].wait()

    return k(v, p0, p1)


def _combine_body(v0_ref, v1_ref, g_ref, y_ref):
    g0 = g_ref[:, 0:1]
    g1 = g_ref[:, 1:2]
    s = g0 * jnp.exp(v0_ref[...]) + g1 * jnp.exp(v1_ref[...])
    s = jnp.where(s == 0.0, jnp.float32(_EPS), s)
    y_ref[...] = jnp.log(s)


def _combine(v0, v1, g):
    rb = 512
    return pl.pallas_call(
        _combine_body,
        grid=(N_TOK // rb,),
        in_specs=[
            pl.BlockSpec((rb, D_MODEL), lambda i: (i, 0)),
            pl.BlockSpec((rb, D_MODEL), lambda i: (i, 0)),
            pl.BlockSpec((rb, 2), lambda i: (i, 0)),
        ],
        out_specs=pl.BlockSpec((rb, D_MODEL), lambda i: (i, 0)),
        out_shape=jax.ShapeDtypeStruct((N_TOK, D_MODEL), jnp.float32),
    )(v0, v1, g)


def kernel(x, w_gate, fc1, b1, fc2, b2):
    wg_pad = jnp.pad(w_gate, ((0, 0), (0, LANES - NE)))
    pos, g, binfo, loss = _routing(x, wg_pad)
    p0 = pos[:, 0].reshape(-1, SCCH)
    p1 = pos[:, 1].reshape(-1, SCCH)
    x_sorted = _dispatch(x, p0, p1)
    v = _gmm(binfo[0], x_sorted,
             fc1.astype(jnp.bfloat16), b1[:, None, :],
             fc2.astype(jnp.bfloat16), b2[:, None, :])
    v0, v1 = _combine_gather(v, p0, p1)
    y = _combine(v0, v1, g)
    return y, loss[0, 0]
